# baseline (speedup 1.0000x reference)
import functools

import jax
import jax.numpy as jnp
from jax import lax
from jax.experimental import pallas as pl
from jax.experimental.pallas import tpu as pltpu
from jax.experimental.pallas import tpu_sc as plsc

_B = 64
_S = 4096
_D = 256
_V = 6
_BOT = 8
_K = max(1, int(_S * 0.1))
_KPAD = 416
_LANES = 16
_CHUNKS = _S // _LANES
_NWORKERS = 32
_BPW = _B // _NWORKERS


def _sigmoid(x):
    return 1.0 / (1.0 + jnp.exp(-x))


def _scorer_body(tab_ref, w1_ref, b1_ref, w2_ref, b2_ref,
                 stab_ref, vslot_ref, p_ref):
    f32 = jnp.float32
    h = lax.dot_general(tab_ref[...], w1_ref[...], (((1,), (1,)), ((), ())),
                        preferred_element_type=f32, precision=lax.Precision.HIGHEST)
    h = jnp.maximum(h + b1_ref[...], 0.0)
    logit = jnp.sum(h * w2_ref[...], axis=1, keepdims=True) + b2_ref[...]
    scol = _sigmoid(logit)
    eye = jnp.eye(8, dtype=f32)
    tr = lambda col: lax.dot_general(col, eye, (((0,), (0,)), ((), ())),
                                     preferred_element_type=f32,
                                     precision=lax.Precision.HIGHEST)
    srow = tr(scol)
    scol_b = jnp.broadcast_to(scol, (8, 8))
    srow_b = jnp.broadcast_to(srow, (8, 8))
    lane = lax.broadcasted_iota(jnp.int32, (8, 8), 1)
    subl = lax.broadcasted_iota(jnp.int32, (8, 8), 0)
    validj = lane < _V
    eqlow = (srow_b == scol_b) & (lane < subl) & validj
    first_col = jnp.sum(jnp.where(eqlow, 1.0, 0.0), axis=1, keepdims=True) == 0.0
    first_row = tr(jnp.where(first_col, 1.0, 0.0))
    first_row_b = jnp.broadcast_to(first_row, (8, 8))
    gt = (srow_b > scol_b) & validj
    gid_col = jnp.sum(jnp.where(gt, 1.0, 0.0) * first_row_b, axis=1, keepdims=True)
    gid_row = tr(gid_col)
    gid_row_b = jnp.broadcast_to(gid_row, (8, 8))
    p_ref[...] = jnp.where(
        (gid_row_b == subl.astype(f32)) & (first_row_b > 0.0) & validj, 1.0, 0.0)
    pad8 = jnp.zeros((1, 8), f32)
    stab_ref[...] = jnp.concatenate([srow, pad8], axis=1)
    vslot_ref[...] = jnp.concatenate(
        [gid_row, pad8], axis=1).astype(jnp.int32)


def _finish_body(cnt_ref, p_ref, tab_ref, wq_ref, wsum_ref, bm1_ref,
                 wm2_ref, bm2_ref, wm3_ref, bm3_ref, wc1_ref, bc1_ref,
                 wc2_ref, bc2_ref, out_ref):
    f32 = jnp.float32
    hi = lax.Precision.HIGHEST
    cnt8 = cnt_ref[...][:, :8].astype(f32)
    ltri = jnp.where(
        lax.broadcasted_iota(jnp.int32, (8, 8), 0)
        < lax.broadcasted_iota(jnp.int32, (8, 8), 1), 1.0, 0.0)
    base = lax.dot_general(cnt8, ltri, (((1,), (0,)), ((), ())),
                           preferred_element_type=f32, precision=hi)
    sel8 = jnp.minimum(jnp.maximum(float(_K) - base, 0.0), cnt8)
    tabs = lax.dot_general(p_ref[...], tab_ref[...], (((1,), (0,)), ((), ())),
                           preferred_element_type=f32, precision=hi)
    query = lax.dot_general(cnt8, tabs, (((1,), (0,)), ((), ())),
                            preferred_element_type=f32, precision=hi) * (1.0 / _S)
    qa = lax.dot_general(query, wq_ref[...], (((1,), (1,)), ((), ())),
                         preferred_element_type=f32, precision=hi)
    tb = lax.dot_general(tabs, wsum_ref[...], (((1,), (1,)), ((), ())),
                         preferred_element_type=f32, precision=hi)
    m1 = jnp.maximum(qa[:, None, :] + tb[None, :, :] + bm1_ref[...][None], 0.0)
    m1 = m1.reshape(_B * 8, _BOT)
    m2 = lax.dot_general(m1, wm2_ref[...], (((1,), (1,)), ((), ())),
                         preferred_element_type=f32, precision=lax.Precision.HIGHEST)
    m2 = jnp.maximum(m2 + bm2_ref[...], 0.0)
    m3 = lax.dot_general(m2, wm3_ref[...], (((1,), (1,)), ((), ())),
                         preferred_element_type=f32, precision=lax.Precision.HIGHEST) + bm3_ref[...]
    m3 = m3.reshape(_B, 8, _D)
    approx = jnp.sum(sel8[:, :, None] * m3, axis=1) * (1.0 / _K)
    c = lax.dot_general(approx, wc1_ref[...], (((1,), (1,)), ((), ())),
                        preferred_element_type=f32, precision=lax.Precision.HIGHEST)
    c = jnp.maximum(c + bc1_ref[...], 0.0)
    logit = jnp.sum(c * wc2_ref[...], axis=1, keepdims=True) + bc2_ref[...]
    out_ref[...] = _sigmoid(jnp.broadcast_to(logit, (_B, 128)))


@functools.lru_cache(maxsize=1)
def _make_sc_kernel():
    mesh = plsc.VectorSubcoreMesh(core_axis_name="c", subcore_axis_name="s")

    @functools.partial(
        pl.kernel,
        mesh=mesh,
        compiler_params=pltpu.CompilerParams(needs_layout_passes=False),
        out_type=[
            jax.ShapeDtypeStruct((_B, _S), jnp.float32),
            jax.ShapeDtypeStruct((_B, _KPAD), jnp.int32),
            jax.ShapeDtypeStruct((_B, _LANES), jnp.int32),
        ],
        scratch_types=[
            pltpu.VMEM((_S,), jnp.int32),
            pltpu.VMEM((_S,), jnp.float32),
            pltpu.VMEM((_S,), jnp.int32),
            pltpu.VMEM((_KPAD,), jnp.int32),
            pltpu.VMEM((_S,), jnp.int32),
            pltpu.VMEM((_LANES,), jnp.float32),
            pltpu.VMEM((_LANES,), jnp.int32),
            pltpu.VMEM((_LANES,), jnp.int32),
        ],
    )
    def sc_kernel(x_hbm, stab_hbm, vslot_hbm,
                  scores_hbm, idx_hbm, cnt_hbm,
                  x_v, sc_v, g_v, idx_v, hc_v, stab_v, vslot_v, cnt_v):
        pltpu.sync_copy(stab_hbm.at[0], stab_v)
        pltpu.sync_copy(vslot_hbm.at[0], vslot_v)
        wid = lax.axis_index("s") * 2 + lax.axis_index("c")
        iota = lax.iota(jnp.int32, _LANES)
        zero = jnp.zeros((_LANES,), jnp.int32)
        UNROLL = 4

        for bl in range(_BPW):
            b = wid * _BPW + bl
            pltpu.sync_copy(x_hbm.at[b], x_v)

            def pass1(c, carry):
                for u in range(UNROLL):
                    off = (c * UNROLL + u) * _LANES
                    hc_v[pl.ds(off, _LANES)] = zero
                    x16 = x_v[pl.ds(off, _LANES)]
                    s16 = plsc.load_gather(stab_v, [x16])
                    g16 = plsc.load_gather(vslot_v, [x16])
                    sc_v[pl.ds(off, _LANES)] = s16
                    g_v[pl.ds(off, _LANES)] = g16
                    dup, lastm = plsc.scan_count(g16)
                    plsc.store_scatter(hc_v, [off + g16], dup, mask=lastm)
                return carry

            lax.fori_loop(0, _CHUNKS // UNROLL, pass1, 0)

            def prefix(c, run):
                for u in range(UNROLL):
                    off = (c * UNROLL + u) * _LANES
                    h = hc_v[pl.ds(off, _LANES)]
                    hc_v[pl.ds(off, _LANES)] = run
                    run = run + h
                return run

            tot = lax.fori_loop(0, _CHUNKS // UNROLL, prefix, zero)
            base = plsc.cumsum(tot) - tot

            def foldbase(c, carry):
                for u in range(UNROLL):
                    off = (c * UNROLL + u) * _LANES
                    hc_v[pl.ds(off, _LANES)] = hc_v[pl.ds(off, _LANES)] + base
                return carry

            lax.fori_loop(0, _CHUNKS // UNROLL, foldbase, 0)

            def pass2(c, carry):
                for u in range(UNROLL):
                    off = (c * UNROLL + u) * _LANES
                    g16 = g_v[pl.ds(off, _LANES)]
                    dup, _ = plsc.scan_count(g16)
                    prior = plsc.load_gather(hc_v, [off + g16])
                    rank = prior + dup - 1
                    pos = off + iota
                    plsc.store_scatter(idx_v, [rank], pos, mask=rank < _K)
                return carry

            lax.fori_loop(0, _CHUNKS // UNROLL, pass2, 0)

            pltpu.sync_copy(sc_v, scores_hbm.at[b])
            pltpu.sync_copy(idx_v, idx_hbm.at[b])
            cnt_v[...] = tot
            pltpu.sync_copy(cnt_v, cnt_hbm.at[b])

    return sc_kernel


def _sc_call(x32, stab16, usc16):
    return _make_sc_kernel()(x32, stab16, usc16)


def kernel(x, table, W1, b1, W2, b2, Wm1, bm1, Wm2, bm2, Wm3, bm3,
           Wc1, bc1, Wc2, bc2):
    f32 = jnp.float32
    x32 = x.astype(jnp.int32)
    tab8 = jnp.concatenate([table, jnp.zeros((8 - _V, _D), f32)], axis=0)

    stab16, vslot16, pmat = pl.pallas_call(
        _scorer_body,
        out_shape=[
            jax.ShapeDtypeStruct((1, _LANES), f32),
            jax.ShapeDtypeStruct((1, _LANES), jnp.int32),
            jax.ShapeDtypeStruct((8, 8), f32),
        ],
    )(tab8, W1, b1.reshape(1, -1), W2, b2.reshape(1, 1))

    scores, idx_pad, cnt_slot = _sc_call(x32, stab16, vslot16)
    idx = idx_pad[:, :_K]

    wq = Wm1[:, :_D]
    wsum = Wm1[:, _D:2 * _D] + Wm1[:, 2 * _D:]
    out_mat = pl.pallas_call(
        _finish_body,
        out_shape=jax.ShapeDtypeStruct((_B, 128), f32),
    )(cnt_slot, pmat, tab8, wq, wsum, bm1.reshape(1, -1), Wm2,
      bm2.reshape(1, -1), Wm3, bm3.reshape(1, -1), Wc1, bc1.reshape(1, -1),
      Wc2, bc2.reshape(1, 1))
    out = out_mat[:, 0]
    return (out, idx, scores)


# speedup vs baseline: 1.3455x; 1.3455x over previous
import functools

import jax
import jax.numpy as jnp
from jax import lax
from jax.experimental import pallas as pl
from jax.experimental.pallas import tpu as pltpu
from jax.experimental.pallas import tpu_sc as plsc

_B = 64
_S = 4096
_D = 256
_V = 6
_BOT = 8
_K = max(1, int(_S * 0.1))
_KPAD = 416
_LANES = 16
_CHUNKS = _S // _LANES
_NWORKERS = 32
_BPW = _B // _NWORKERS


def _sigmoid(x):
    return 1.0 / (1.0 + jnp.exp(-x))


def _scorer_body(tab_ref, w1_ref, b1_ref, w2_ref, b2_ref,
                 stab_ref, vslot_ref, p_ref):
    f32 = jnp.float32
    h = lax.dot_general(tab_ref[...], w1_ref[...], (((1,), (1,)), ((), ())),
                        preferred_element_type=f32, precision=lax.Precision.HIGHEST)
    h = jnp.maximum(h + b1_ref[...], 0.0)
    logit = jnp.sum(h * w2_ref[...], axis=1, keepdims=True) + b2_ref[...]
    scol = _sigmoid(logit)
    eye = jnp.eye(8, dtype=f32)
    tr = lambda col: lax.dot_general(col, eye, (((0,), (0,)), ((), ())),
                                     preferred_element_type=f32,
                                     precision=lax.Precision.HIGHEST)
    srow = tr(scol)
    scol_b = jnp.broadcast_to(scol, (8, 8))
    srow_b = jnp.broadcast_to(srow, (8, 8))
    lane = lax.broadcasted_iota(jnp.int32, (8, 8), 1)
    subl = lax.broadcasted_iota(jnp.int32, (8, 8), 0)
    validj = lane < _V
    eqlow = (srow_b == scol_b) & (lane < subl) & validj
    first_col = jnp.sum(jnp.where(eqlow, 1.0, 0.0), axis=1, keepdims=True) == 0.0
    first_row = tr(jnp.where(first_col, 1.0, 0.0))
    first_row_b = jnp.broadcast_to(first_row, (8, 8))
    gt = (srow_b > scol_b) & validj
    gid_col = jnp.sum(jnp.where(gt, 1.0, 0.0) * first_row_b, axis=1, keepdims=True)
    gid_row = tr(gid_col)
    gid_row_b = jnp.broadcast_to(gid_row, (8, 8))
    p_ref[...] = jnp.where(
        (gid_row_b == subl.astype(f32)) & (first_row_b > 0.0) & validj, 1.0, 0.0)
    pad8 = jnp.zeros((1, 8), f32)
    stab_ref[...] = jnp.concatenate([srow, pad8], axis=1)
    vslot_ref[...] = jnp.concatenate(
        [gid_row, pad8], axis=1).astype(jnp.int32)


def _finish_body(cnt_ref, p_ref, tab_ref, wq_ref, wsum_ref, bm1_ref,
                 wm2_ref, bm2_ref, wm3_ref, bm3_ref, wc1_ref, bc1_ref,
                 wc2_ref, bc2_ref, out_ref):
    f32 = jnp.float32
    hi = lax.Precision.HIGHEST
    cnt8 = cnt_ref[...][:, :8].astype(f32)
    ltri = jnp.where(
        lax.broadcasted_iota(jnp.int32, (8, 8), 0)
        < lax.broadcasted_iota(jnp.int32, (8, 8), 1), 1.0, 0.0)
    base = lax.dot_general(cnt8, ltri, (((1,), (0,)), ((), ())),
                           preferred_element_type=f32, precision=hi)
    sel8 = jnp.minimum(jnp.maximum(float(_K) - base, 0.0), cnt8)
    tabs = lax.dot_general(p_ref[...], tab_ref[...], (((1,), (0,)), ((), ())),
                           preferred_element_type=f32, precision=hi)
    query = lax.dot_general(cnt8, tabs, (((1,), (0,)), ((), ())),
                            preferred_element_type=f32, precision=hi) * (1.0 / _S)
    qa = lax.dot_general(query, wq_ref[...], (((1,), (1,)), ((), ())),
                         preferred_element_type=f32, precision=hi)
    tb = lax.dot_general(tabs, wsum_ref[...], (((1,), (1,)), ((), ())),
                         preferred_element_type=f32, precision=hi)
    m1 = jnp.maximum(qa[:, None, :] + tb[None, :, :] + bm1_ref[...][None], 0.0)
    m1 = m1.reshape(_B * 8, _BOT)
    m2 = lax.dot_general(m1, wm2_ref[...], (((1,), (1,)), ((), ())),
                         preferred_element_type=f32, precision=lax.Precision.HIGHEST)
    m2 = jnp.maximum(m2 + bm2_ref[...], 0.0)
    m3 = lax.dot_general(m2, wm3_ref[...], (((1,), (1,)), ((), ())),
                         preferred_element_type=f32, precision=lax.Precision.HIGHEST) + bm3_ref[...]
    m3 = m3.reshape(_B, 8, _D)
    approx = jnp.sum(sel8[:, :, None] * m3, axis=1) * (1.0 / _K)
    c = lax.dot_general(approx, wc1_ref[...], (((1,), (1,)), ((), ())),
                        preferred_element_type=f32, precision=lax.Precision.HIGHEST)
    c = jnp.maximum(c + bc1_ref[...], 0.0)
    logit = jnp.sum(c * wc2_ref[...], axis=1, keepdims=True) + bc2_ref[...]
    out_ref[...] = _sigmoid(jnp.broadcast_to(logit, (_B, 128)))


@functools.lru_cache(maxsize=1)
def _make_sc_kernel():
    mesh = plsc.VectorSubcoreMesh(core_axis_name="c", subcore_axis_name="s")

    @functools.partial(
        pl.kernel,
        mesh=mesh,
        compiler_params=pltpu.CompilerParams(needs_layout_passes=False),
        out_type=[
            jax.ShapeDtypeStruct((_B, _S), jnp.float32),
            jax.ShapeDtypeStruct((_B, _KPAD), jnp.int32),
            jax.ShapeDtypeStruct((_B, _LANES), jnp.int32),
        ],
        scratch_types=[
            pltpu.VMEM((_S,), jnp.int32),
            pltpu.VMEM((_S,), jnp.float32),
            pltpu.VMEM((_S,), jnp.int32),
            pltpu.VMEM((_KPAD,), jnp.int32),
            pltpu.VMEM((_S,), jnp.int32),
            pltpu.VMEM((_LANES,), jnp.float32),
            pltpu.VMEM((_LANES,), jnp.int32),
            pltpu.VMEM((_LANES,), jnp.int32),
        ],
    )
    def sc_kernel(x_hbm, stab_hbm, vslot_hbm,
                  scores_hbm, idx_hbm, cnt_hbm,
                  x_v, sc_v, g_v, idx_v, hc_v, stab_v, vslot_v, cnt_v):
        pltpu.sync_copy(stab_hbm.at[0], stab_v)
        pltpu.sync_copy(vslot_hbm.at[0], vslot_v)
        wid = lax.axis_index("s") * 2 + lax.axis_index("c")
        iota = lax.iota(jnp.int32, _LANES)
        zero = jnp.zeros((_LANES,), jnp.int32)
        UNROLL = 4

        for bl in range(_BPW):
            b = wid * _BPW + bl
            pltpu.sync_copy(x_hbm.at[b], x_v)

            def pass1(c, carry):
                offs = [(c * UNROLL + u) * _LANES for u in range(UNROLL)]
                for off in offs:
                    hc_v[pl.ds(off, _LANES)] = zero
                xs = [x_v[pl.ds(off, _LANES)] for off in offs]
                ss = [plsc.load_gather(stab_v, [x16]) for x16 in xs]
                gs = [plsc.load_gather(vslot_v, [x16]) for x16 in xs]
                for off, s16 in zip(offs, ss):
                    sc_v[pl.ds(off, _LANES)] = s16
                for off, g16 in zip(offs, gs):
                    g_v[pl.ds(off, _LANES)] = g16
                scans = [plsc.scan_count(g16) for g16 in gs]
                for off, g16, (dup, lastm) in zip(offs, gs, scans):
                    plsc.store_scatter(hc_v, [off + g16], dup, mask=lastm)
                return carry

            lax.fori_loop(0, _CHUNKS // UNROLL, pass1, 0)

            def prefix(c, run):
                offs = [(c * UNROLL + u) * _LANES for u in range(UNROLL)]
                hs = [hc_v[pl.ds(off, _LANES)] for off in offs]
                for off, h in zip(offs, hs):
                    hc_v[pl.ds(off, _LANES)] = run
                    run = run + h
                return run

            tot = lax.fori_loop(0, _CHUNKS // UNROLL, prefix, zero)
            base = plsc.cumsum(tot) - tot

            def foldbase(c, carry):
                offs = [(c * UNROLL + u) * _LANES for u in range(UNROLL)]
                hs = [hc_v[pl.ds(off, _LANES)] for off in offs]
                for off, h in zip(offs, hs):
                    hc_v[pl.ds(off, _LANES)] = h + base
                return carry

            lax.fori_loop(0, _CHUNKS // UNROLL, foldbase, 0)

            def pass2(c, carry):
                offs = [(c * UNROLL + u) * _LANES for u in range(UNROLL)]
                gs = [g_v[pl.ds(off, _LANES)] for off in offs]
                scans = [plsc.scan_count(g16) for g16 in gs]
                priors = [plsc.load_gather(hc_v, [off + g16])
                          for off, g16 in zip(offs, gs)]
                for off, (dup, _), prior in zip(offs, scans, priors):
                    rank = prior + dup - 1
                    pos = off + iota
                    plsc.store_scatter(idx_v, [rank], pos, mask=rank < _K)
                return carry

            lax.fori_loop(0, _CHUNKS // UNROLL, pass2, 0)

            pltpu.sync_copy(sc_v, scores_hbm.at[b])
            pltpu.sync_copy(idx_v, idx_hbm.at[b])
            cnt_v[...] = tot
            pltpu.sync_copy(cnt_v, cnt_hbm.at[b])

    return sc_kernel


def _sc_call(x32, stab16, usc16):
    return _make_sc_kernel()(x32, stab16, usc16)


def kernel(x, table, W1, b1, W2, b2, Wm1, bm1, Wm2, bm2, Wm3, bm3,
           Wc1, bc1, Wc2, bc2):
    f32 = jnp.float32
    x32 = x.astype(jnp.int32)
    tab8 = jnp.concatenate([table, jnp.zeros((8 - _V, _D), f32)], axis=0)

    stab16, vslot16, pmat = pl.pallas_call(
        _scorer_body,
        out_shape=[
            jax.ShapeDtypeStruct((1, _LANES), f32),
            jax.ShapeDtypeStruct((1, _LANES), jnp.int32),
            jax.ShapeDtypeStruct((8, 8), f32),
        ],
    )(tab8, W1, b1.reshape(1, -1), W2, b2.reshape(1, 1))

    scores, idx_pad, cnt_slot = _sc_call(x32, stab16, vslot16)
    idx = idx_pad[:, :_K]

    wq = Wm1[:, :_D]
    wsum = Wm1[:, _D:2 * _D] + Wm1[:, 2 * _D:]
    out_mat = pl.pallas_call(
        _finish_body,
        out_shape=jax.ShapeDtypeStruct((_B, 128), f32),
    )(cnt_slot, pmat, tab8, wq, wsum, bm1.reshape(1, -1), Wm2,
      bm2.reshape(1, -1), Wm3, bm3.reshape(1, -1), Wc1, bc1.reshape(1, -1),
      Wc2, bc2.reshape(1, 1))
    out = out_mat[:, 0]
    return (out, idx, scores)


# speedup vs baseline: 1.4349x; 1.0665x over previous
import functools

import jax
import jax.numpy as jnp
from jax import lax
from jax.experimental import pallas as pl
from jax.experimental.pallas import tpu as pltpu
from jax.experimental.pallas import tpu_sc as plsc

_B = 64
_S = 4096
_D = 256
_V = 6
_BOT = 8
_K = max(1, int(_S * 0.1))
_KPAD = 416
_LANES = 16
_CHUNKS = _S // _LANES
_NWORKERS = 32
_BPW = _B // _NWORKERS


def _sigmoid(x):
    return 1.0 / (1.0 + jnp.exp(-x))


def _scorer_body(tab_ref, w1_ref, b1_ref, w2_ref, b2_ref,
                 stab_ref, vslot_ref, p_ref):
    f32 = jnp.float32
    h = lax.dot_general(tab_ref[...], w1_ref[...], (((1,), (1,)), ((), ())),
                        preferred_element_type=f32, precision=lax.Precision.HIGHEST)
    h = jnp.maximum(h + b1_ref[...], 0.0)
    logit = jnp.sum(h * w2_ref[...], axis=1, keepdims=True) + b2_ref[...]
    scol = _sigmoid(logit)
    eye = jnp.eye(8, dtype=f32)
    tr = lambda col: lax.dot_general(col, eye, (((0,), (0,)), ((), ())),
                                     preferred_element_type=f32,
                                     precision=lax.Precision.HIGHEST)
    srow = tr(scol)
    scol_b = jnp.broadcast_to(scol, (8, 8))
    srow_b = jnp.broadcast_to(srow, (8, 8))
    lane = lax.broadcasted_iota(jnp.int32, (8, 8), 1)
    subl = lax.broadcasted_iota(jnp.int32, (8, 8), 0)
    validj = lane < _V
    eqlow = (srow_b == scol_b) & (lane < subl) & validj
    first_col = jnp.sum(jnp.where(eqlow, 1.0, 0.0), axis=1, keepdims=True) == 0.0
    first_row = tr(jnp.where(first_col, 1.0, 0.0))
    first_row_b = jnp.broadcast_to(first_row, (8, 8))
    gt = (srow_b > scol_b) & validj
    gid_col = jnp.sum(jnp.where(gt, 1.0, 0.0) * first_row_b, axis=1, keepdims=True)
    gid_row = tr(gid_col)
    gid_row_b = jnp.broadcast_to(gid_row, (8, 8))
    p_ref[...] = jnp.where(
        (gid_row_b == subl.astype(f32)) & (first_row_b > 0.0) & validj, 1.0, 0.0)
    pad8 = jnp.zeros((1, 8), f32)
    stab_ref[...] = jnp.concatenate([srow, pad8], axis=1)
    vslot_ref[...] = jnp.concatenate(
        [gid_row, pad8], axis=1).astype(jnp.int32)


def _finish_body(cnt_ref, p_ref, tab_ref, wq_ref, wsum_ref, bm1_ref,
                 wm2_ref, bm2_ref, wm3_ref, bm3_ref, wc1_ref, bc1_ref,
                 wc2_ref, bc2_ref, out_ref):
    f32 = jnp.float32
    hi = lax.Precision.HIGHEST
    cnt8 = cnt_ref[...][:, :8].astype(f32)
    ltri = jnp.where(
        lax.broadcasted_iota(jnp.int32, (8, 8), 0)
        < lax.broadcasted_iota(jnp.int32, (8, 8), 1), 1.0, 0.0)
    base = lax.dot_general(cnt8, ltri, (((1,), (0,)), ((), ())),
                           preferred_element_type=f32, precision=hi)
    sel8 = jnp.minimum(jnp.maximum(float(_K) - base, 0.0), cnt8)
    tabs = lax.dot_general(p_ref[...], tab_ref[...], (((1,), (0,)), ((), ())),
                           preferred_element_type=f32, precision=hi)
    query = lax.dot_general(cnt8, tabs, (((1,), (0,)), ((), ())),
                            preferred_element_type=f32, precision=hi) * (1.0 / _S)
    qa = lax.dot_general(query, wq_ref[...], (((1,), (1,)), ((), ())),
                         preferred_element_type=f32, precision=hi)
    tb = lax.dot_general(tabs, wsum_ref[...], (((1,), (1,)), ((), ())),
                         preferred_element_type=f32, precision=hi)
    m1 = jnp.maximum(qa[:, None, :] + tb[None, :, :] + bm1_ref[...][None], 0.0)
    m1 = m1.reshape(_B * 8, _BOT)
    m2 = lax.dot_general(m1, wm2_ref[...], (((1,), (1,)), ((), ())),
                         preferred_element_type=f32, precision=lax.Precision.HIGHEST)
    m2 = jnp.maximum(m2 + bm2_ref[...], 0.0)
    m3 = lax.dot_general(m2, wm3_ref[...], (((1,), (1,)), ((), ())),
                         preferred_element_type=f32, precision=lax.Precision.HIGHEST) + bm3_ref[...]
    m3 = m3.reshape(_B, 8, _D)
    approx = jnp.sum(sel8[:, :, None] * m3, axis=1) * (1.0 / _K)
    c = lax.dot_general(approx, wc1_ref[...], (((1,), (1,)), ((), ())),
                        preferred_element_type=f32, precision=lax.Precision.HIGHEST)
    c = jnp.maximum(c + bc1_ref[...], 0.0)
    logit = jnp.sum(c * wc2_ref[...], axis=1, keepdims=True) + bc2_ref[...]
    out_ref[...] = _sigmoid(jnp.broadcast_to(logit, (_B, 128)))


@functools.lru_cache(maxsize=1)
def _make_sc_kernel():
    mesh = plsc.VectorSubcoreMesh(core_axis_name="c", subcore_axis_name="s")

    @functools.partial(
        pl.kernel,
        mesh=mesh,
        compiler_params=pltpu.CompilerParams(needs_layout_passes=False),
        out_type=[
            jax.ShapeDtypeStruct((_B, _S), jnp.float32),
            jax.ShapeDtypeStruct((_B, _KPAD), jnp.int32),
            jax.ShapeDtypeStruct((_B, _LANES), jnp.int32),
        ],
        scratch_types=[
            pltpu.VMEM((_S,), jnp.int32),
            pltpu.VMEM((_S,), jnp.float32),
            pltpu.VMEM((_S,), jnp.int32),
            pltpu.VMEM((_KPAD,), jnp.int32),
            pltpu.VMEM((_S,), jnp.int32),
            pltpu.VMEM((_LANES,), jnp.float32),
            pltpu.VMEM((_LANES,), jnp.int32),
            pltpu.VMEM((_LANES,), jnp.int32),
        ],
    )
    def sc_kernel(x_hbm, stab_hbm, vslot_hbm,
                  scores_hbm, idx_hbm, cnt_hbm,
                  x_v, sc_v, g_v, idx_v, hc_v, stab_v, vslot_v, cnt_v):
        pltpu.sync_copy(stab_hbm.at[0], stab_v)
        pltpu.sync_copy(vslot_hbm.at[0], vslot_v)
        wid = lax.axis_index("s") * 2 + lax.axis_index("c")
        iota = lax.iota(jnp.int32, _LANES)
        zero = jnp.zeros((_LANES,), jnp.int32)
        UNROLL = 8

        for bl in range(_BPW):
            b = wid * _BPW + bl
            pltpu.sync_copy(x_hbm.at[b], x_v)

            def pass1(c, carry):
                offs = [(c * UNROLL + u) * _LANES for u in range(UNROLL)]
                for off in offs:
                    hc_v[pl.ds(off, _LANES)] = zero
                xs = [x_v[pl.ds(off, _LANES)] for off in offs]
                ss = [plsc.load_gather(stab_v, [x16]) for x16 in xs]
                gs = [plsc.load_gather(vslot_v, [x16]) for x16 in xs]
                for off, s16 in zip(offs, ss):
                    sc_v[pl.ds(off, _LANES)] = s16
                for off, g16 in zip(offs, gs):
                    g_v[pl.ds(off, _LANES)] = g16
                scans = [plsc.scan_count(g16) for g16 in gs]
                for off, g16, (dup, lastm) in zip(offs, gs, scans):
                    plsc.store_scatter(hc_v, [off + g16], dup, mask=lastm)
                return carry

            lax.fori_loop(0, _CHUNKS // UNROLL, pass1, 0)

            def prefix(c, run):
                offs = [(c * UNROLL + u) * _LANES for u in range(UNROLL)]
                hs = [hc_v[pl.ds(off, _LANES)] for off in offs]
                for off, h in zip(offs, hs):
                    hc_v[pl.ds(off, _LANES)] = run
                    run = run + h
                return run

            tot = lax.fori_loop(0, _CHUNKS // UNROLL, prefix, zero)
            base = plsc.cumsum(tot) - tot

            def foldbase(c, carry):
                offs = [(c * UNROLL + u) * _LANES for u in range(UNROLL)]
                hs = [hc_v[pl.ds(off, _LANES)] for off in offs]
                for off, h in zip(offs, hs):
                    hc_v[pl.ds(off, _LANES)] = h + base
                return carry

            lax.fori_loop(0, _CHUNKS // UNROLL, foldbase, 0)

            def pass2(c, carry):
                offs = [(c * UNROLL + u) * _LANES for u in range(UNROLL)]
                gs = [g_v[pl.ds(off, _LANES)] for off in offs]
                scans = [plsc.scan_count(g16) for g16 in gs]
                priors = [plsc.load_gather(hc_v, [off + g16])
                          for off, g16 in zip(offs, gs)]
                for off, (dup, _), prior in zip(offs, scans, priors):
                    rank = prior + dup - 1
                    pos = off + iota
                    plsc.store_scatter(idx_v, [rank], pos, mask=rank < _K)
                return carry

            lax.fori_loop(0, _CHUNKS // UNROLL, pass2, 0)

            pltpu.sync_copy(sc_v, scores_hbm.at[b])
            pltpu.sync_copy(idx_v, idx_hbm.at[b])
            cnt_v[...] = tot
            pltpu.sync_copy(cnt_v, cnt_hbm.at[b])

    return sc_kernel


def _sc_call(x32, stab16, usc16):
    return _make_sc_kernel()(x32, stab16, usc16)


def kernel(x, table, W1, b1, W2, b2, Wm1, bm1, Wm2, bm2, Wm3, bm3,
           Wc1, bc1, Wc2, bc2):
    f32 = jnp.float32
    x32 = x.astype(jnp.int32)
    tab8 = jnp.concatenate([table, jnp.zeros((8 - _V, _D), f32)], axis=0)

    stab16, vslot16, pmat = pl.pallas_call(
        _scorer_body,
        out_shape=[
            jax.ShapeDtypeStruct((1, _LANES), f32),
            jax.ShapeDtypeStruct((1, _LANES), jnp.int32),
            jax.ShapeDtypeStruct((8, 8), f32),
        ],
    )(tab8, W1, b1.reshape(1, -1), W2, b2.reshape(1, 1))

    scores, idx_pad, cnt_slot = _sc_call(x32, stab16, vslot16)
    idx = idx_pad[:, :_K]

    wq = Wm1[:, :_D]
    wsum = Wm1[:, _D:2 * _D] + Wm1[:, 2 * _D:]
    out_mat = pl.pallas_call(
        _finish_body,
        out_shape=jax.ShapeDtypeStruct((_B, 128), f32),
    )(cnt_slot, pmat, tab8, wq, wsum, bm1.reshape(1, -1), Wm2,
      bm2.reshape(1, -1), Wm3, bm3.reshape(1, -1), Wc1, bc1.reshape(1, -1),
      Wc2, bc2.reshape(1, 1))
    out = out_mat[:, 0]
    return (out, idx, scores)


# speedup vs baseline: 1.4858x; 1.0355x over previous
import functools

import jax
import jax.numpy as jnp
from jax import lax
from jax.experimental import pallas as pl
from jax.experimental.pallas import tpu as pltpu
from jax.experimental.pallas import tpu_sc as plsc

_B = 64
_S = 4096
_D = 256
_V = 6
_BOT = 8
_K = max(1, int(_S * 0.1))
_KPAD = 416
_LANES = 16
_CHUNKS = _S // _LANES
_NWORKERS = 32
_BPW = _B // _NWORKERS


def _sigmoid(x):
    return 1.0 / (1.0 + jnp.exp(-x))


def _scorer_body(tab_ref, w1_ref, b1_ref, w2_ref, b2_ref,
                 stab_ref, vslot_ref, p_ref):
    f32 = jnp.float32
    h = lax.dot_general(tab_ref[...], w1_ref[...], (((1,), (1,)), ((), ())),
                        preferred_element_type=f32, precision=lax.Precision.HIGHEST)
    h = jnp.maximum(h + b1_ref[...], 0.0)
    logit = jnp.sum(h * w2_ref[...], axis=1, keepdims=True) + b2_ref[...]
    scol = _sigmoid(logit)
    scol = jnp.concatenate([scol, jnp.zeros((8 - _V, 1), f32)], axis=0)
    eye = jnp.eye(8, dtype=f32)
    tr = lambda col: lax.dot_general(col, eye, (((0,), (0,)), ((), ())),
                                     preferred_element_type=f32,
                                     precision=lax.Precision.HIGHEST)
    srow = tr(scol)
    scol_b = jnp.broadcast_to(scol, (8, 8))
    srow_b = jnp.broadcast_to(srow, (8, 8))
    lane = lax.broadcasted_iota(jnp.int32, (8, 8), 1)
    subl = lax.broadcasted_iota(jnp.int32, (8, 8), 0)
    validj = lane < _V
    eqlow = (srow_b == scol_b) & (lane < subl) & validj
    first_col = jnp.sum(jnp.where(eqlow, 1.0, 0.0), axis=1, keepdims=True) == 0.0
    first_row = tr(jnp.where(first_col, 1.0, 0.0))
    first_row_b = jnp.broadcast_to(first_row, (8, 8))
    gt = (srow_b > scol_b) & validj
    gid_col = jnp.sum(jnp.where(gt, 1.0, 0.0) * first_row_b, axis=1, keepdims=True)
    gid_row = tr(gid_col)
    gid_row_b = jnp.broadcast_to(gid_row, (8, 8))
    p_ref[...] = jnp.where(
        (gid_row_b == subl.astype(f32)) & (first_row_b > 0.0) & validj, 1.0, 0.0)
    pad8 = jnp.zeros((1, 8), f32)
    stab_ref[...] = jnp.concatenate([srow, pad8], axis=1)
    vslot_ref[...] = jnp.concatenate(
        [gid_row, pad8], axis=1).astype(jnp.int32)


def _finish_body(cnt_ref, p_ref, tab_ref, wm1_ref, bm1_ref,
                 wm2_ref, bm2_ref, wm3_ref, bm3_ref, wc1_ref, bc1_ref,
                 wc2_ref, bc2_ref, out_ref):
    f32 = jnp.float32
    hi = lax.Precision.HIGHEST
    wm1 = wm1_ref[...]
    wq = wm1[:, :_D]
    wsum = wm1[:, _D:2 * _D] + wm1[:, 2 * _D:]
    cnt8 = cnt_ref[...][:, :8].astype(f32)
    ltri = jnp.where(
        lax.broadcasted_iota(jnp.int32, (8, 8), 0)
        < lax.broadcasted_iota(jnp.int32, (8, 8), 1), 1.0, 0.0)
    base = lax.dot_general(cnt8, ltri, (((1,), (0,)), ((), ())),
                           preferred_element_type=f32, precision=hi)
    sel8 = jnp.minimum(jnp.maximum(float(_K) - base, 0.0), cnt8)
    tabs = lax.dot_general(p_ref[...][:, :_V], tab_ref[...],
                           (((1,), (0,)), ((), ())),
                           preferred_element_type=f32, precision=hi)
    query = lax.dot_general(cnt8, tabs, (((1,), (0,)), ((), ())),
                            preferred_element_type=f32, precision=hi) * (1.0 / _S)
    qa = lax.dot_general(query, wq, (((1,), (1,)), ((), ())),
                         preferred_element_type=f32, precision=hi)
    tb = lax.dot_general(tabs, wsum, (((1,), (1,)), ((), ())),
                         preferred_element_type=f32, precision=hi)
    m1 = jnp.maximum(qa[:, None, :] + tb[None, :, :] + bm1_ref[...], 0.0)
    m1 = m1.reshape(_B * 8, _BOT)
    m2 = lax.dot_general(m1, wm2_ref[...], (((1,), (1,)), ((), ())),
                         preferred_element_type=f32, precision=lax.Precision.HIGHEST)
    m2 = jnp.maximum(m2 + bm2_ref[...], 0.0)
    m3 = lax.dot_general(m2, wm3_ref[...], (((1,), (1,)), ((), ())),
                         preferred_element_type=f32, precision=lax.Precision.HIGHEST) + bm3_ref[...]
    m3 = m3.reshape(_B, 8, _D)
    approx = jnp.sum(sel8[:, :, None] * m3, axis=1) * (1.0 / _K)
    c = lax.dot_general(approx, wc1_ref[...], (((1,), (1,)), ((), ())),
                        preferred_element_type=f32, precision=lax.Precision.HIGHEST)
    c = jnp.maximum(c + bc1_ref[...], 0.0)
    logit = jnp.sum(c * wc2_ref[...], axis=1, keepdims=True) + bc2_ref[...]
    out_ref[...] = _sigmoid(jnp.broadcast_to(logit, (_B, 128)))


@functools.lru_cache(maxsize=1)
def _make_sc_kernel():
    mesh = plsc.VectorSubcoreMesh(core_axis_name="c", subcore_axis_name="s")

    @functools.partial(
        pl.kernel,
        mesh=mesh,
        compiler_params=pltpu.CompilerParams(needs_layout_passes=False),
        out_type=[
            jax.ShapeDtypeStruct((_B, _S), jnp.float32),
            jax.ShapeDtypeStruct((_B, _KPAD), jnp.int32),
            jax.ShapeDtypeStruct((_B, _LANES), jnp.int32),
        ],
        scratch_types=[
            pltpu.VMEM((_S,), jnp.int32),
            pltpu.VMEM((_S,), jnp.float32),
            pltpu.VMEM((_S,), jnp.int32),
            pltpu.VMEM((_KPAD,), jnp.int32),
            pltpu.VMEM((_S,), jnp.int32),
            pltpu.VMEM((_LANES,), jnp.float32),
            pltpu.VMEM((_LANES,), jnp.int32),
            pltpu.VMEM((_LANES,), jnp.int32),
        ],
    )
    def sc_kernel(x_hbm, stab_hbm, vslot_hbm,
                  scores_hbm, idx_hbm, cnt_hbm,
                  x_v, sc_v, g_v, idx_v, hc_v, stab_v, vslot_v, cnt_v):
        pltpu.sync_copy(stab_hbm.at[0], stab_v)
        pltpu.sync_copy(vslot_hbm.at[0], vslot_v)
        wid = lax.axis_index("s") * 2 + lax.axis_index("c")
        iota = lax.iota(jnp.int32, _LANES)
        zero = jnp.zeros((_LANES,), jnp.int32)
        UNROLL = 8

        for bl in range(_BPW):
            b = wid * _BPW + bl
            pltpu.sync_copy(x_hbm.at[b], x_v)

            def pass1(c, carry):
                offs = [(c * UNROLL + u) * _LANES for u in range(UNROLL)]
                for off in offs:
                    hc_v[pl.ds(off, _LANES)] = zero
                xs = [x_v[pl.ds(off, _LANES)] for off in offs]
                ss = [plsc.load_gather(stab_v, [x16]) for x16 in xs]
                gs = [plsc.load_gather(vslot_v, [x16]) for x16 in xs]
                for off, s16 in zip(offs, ss):
                    sc_v[pl.ds(off, _LANES)] = s16
                for off, g16 in zip(offs, gs):
                    g_v[pl.ds(off, _LANES)] = g16
                scans = [plsc.scan_count(g16) for g16 in gs]
                for off, g16, (dup, lastm) in zip(offs, gs, scans):
                    plsc.store_scatter(hc_v, [off + g16], dup, mask=lastm)
                return carry

            lax.fori_loop(0, _CHUNKS // UNROLL, pass1, 0)

            def prefix(c, run):
                offs = [(c * UNROLL + u) * _LANES for u in range(UNROLL)]
                hs = [hc_v[pl.ds(off, _LANES)] for off in offs]
                for off, h in zip(offs, hs):
                    hc_v[pl.ds(off, _LANES)] = run
                    run = run + h
                return run

            tot = lax.fori_loop(0, _CHUNKS // UNROLL, prefix, zero)
            base = plsc.cumsum(tot) - tot

            def foldbase(c, carry):
                offs = [(c * UNROLL + u) * _LANES for u in range(UNROLL)]
                hs = [hc_v[pl.ds(off, _LANES)] for off in offs]
                for off, h in zip(offs, hs):
                    hc_v[pl.ds(off, _LANES)] = h + base
                return carry

            lax.fori_loop(0, _CHUNKS // UNROLL, foldbase, 0)

            def pass2(c, carry):
                offs = [(c * UNROLL + u) * _LANES for u in range(UNROLL)]
                gs = [g_v[pl.ds(off, _LANES)] for off in offs]
                scans = [plsc.scan_count(g16) for g16 in gs]
                priors = [plsc.load_gather(hc_v, [off + g16])
                          for off, g16 in zip(offs, gs)]
                for off, (dup, _), prior in zip(offs, scans, priors):
                    rank = prior + dup - 1
                    pos = off + iota
                    plsc.store_scatter(idx_v, [rank], pos, mask=rank < _K)
                return carry

            lax.fori_loop(0, _CHUNKS // UNROLL, pass2, 0)

            pltpu.sync_copy(sc_v, scores_hbm.at[b])
            pltpu.sync_copy(idx_v, idx_hbm.at[b])
            cnt_v[...] = tot
            pltpu.sync_copy(cnt_v, cnt_hbm.at[b])

    return sc_kernel


def _sc_call(x32, stab16, usc16):
    return _make_sc_kernel()(x32, stab16, usc16)


def kernel(x, table, W1, b1, W2, b2, Wm1, bm1, Wm2, bm2, Wm3, bm3,
           Wc1, bc1, Wc2, bc2):
    f32 = jnp.float32
    x32 = x.astype(jnp.int32)

    stab16, vslot16, pmat = pl.pallas_call(
        _scorer_body,
        out_shape=[
            jax.ShapeDtypeStruct((1, _LANES), f32),
            jax.ShapeDtypeStruct((1, _LANES), jnp.int32),
            jax.ShapeDtypeStruct((8, 8), f32),
        ],
    )(table, W1, b1, W2, b2)

    scores, idx_pad, cnt_slot = _sc_call(x32, stab16, vslot16)
    idx = idx_pad[:, :_K]

    out_mat = pl.pallas_call(
        _finish_body,
        out_shape=jax.ShapeDtypeStruct((_B, 128), f32),
    )(cnt_slot, pmat, table, Wm1, bm1, Wm2, bm2, Wm3, bm3, Wc1, bc1, Wc2, bc2)
    out = out_mat[:, 0]
    return (out, idx, scores)


# speedup vs baseline: 1.4882x; 1.0016x over previous
import functools

import jax
import jax.numpy as jnp
from jax import lax
from jax.experimental import pallas as pl
from jax.experimental.pallas import tpu as pltpu
from jax.experimental.pallas import tpu_sc as plsc

_B = 64
_S = 4096
_D = 256
_V = 6
_BOT = 8
_K = max(1, int(_S * 0.1))
_KPAD = 416
_LANES = 16
_CHUNKS = _S // _LANES
_NWORKERS = 32
_BPW = _B // _NWORKERS


def _sigmoid(x):
    return 1.0 / (1.0 + jnp.exp(-x))


def _scorer_body(tab_ref, w1_ref, b1_ref, w2_ref, b2_ref,
                 stab_ref, vslot_ref, p_ref):
    f32 = jnp.float32
    h = lax.dot_general(tab_ref[...], w1_ref[...], (((1,), (1,)), ((), ())),
                        preferred_element_type=f32, precision=lax.Precision.HIGHEST)
    h = jnp.maximum(h + b1_ref[...], 0.0)
    logit = jnp.sum(h * w2_ref[...], axis=1, keepdims=True) + b2_ref[...]
    scol = _sigmoid(logit)
    scol = jnp.concatenate([scol, jnp.zeros((8 - _V, 1), f32)], axis=0)
    eye = jnp.eye(8, dtype=f32)
    tr = lambda col: lax.dot_general(col, eye, (((0,), (0,)), ((), ())),
                                     preferred_element_type=f32,
                                     precision=lax.Precision.HIGHEST)
    srow = tr(scol)
    scol_b = jnp.broadcast_to(scol, (8, 8))
    srow_b = jnp.broadcast_to(srow, (8, 8))
    lane = lax.broadcasted_iota(jnp.int32, (8, 8), 1)
    subl = lax.broadcasted_iota(jnp.int32, (8, 8), 0)
    validj = lane < _V
    eqlow = (srow_b == scol_b) & (lane < subl) & validj
    first_col = jnp.sum(jnp.where(eqlow, 1.0, 0.0), axis=1, keepdims=True) == 0.0
    first_row = tr(jnp.where(first_col, 1.0, 0.0))
    first_row_b = jnp.broadcast_to(first_row, (8, 8))
    gt = (srow_b > scol_b) & validj
    gid_col = jnp.sum(jnp.where(gt, 1.0, 0.0) * first_row_b, axis=1, keepdims=True)
    gid_row = tr(gid_col)
    gid_row_b = jnp.broadcast_to(gid_row, (8, 8))
    p_ref[...] = jnp.where(
        (gid_row_b == subl.astype(f32)) & (first_row_b > 0.0) & validj, 1.0, 0.0)
    pad8 = jnp.zeros((1, 8), f32)
    stab_ref[...] = jnp.concatenate([srow, pad8], axis=1)
    vslot_ref[...] = jnp.concatenate(
        [gid_row, pad8], axis=1).astype(jnp.int32)


def _finish_body(cnt_ref, p_ref, tab_ref, wm1_ref, bm1_ref,
                 wm2_ref, bm2_ref, wm3_ref, bm3_ref, wc1_ref, bc1_ref,
                 wc2_ref, bc2_ref, out_ref):
    f32 = jnp.float32
    hi = lax.Precision.HIGHEST
    wm1 = wm1_ref[...]
    wq = wm1[:, :_D]
    wsum = wm1[:, _D:2 * _D] + wm1[:, 2 * _D:]
    cnt8 = cnt_ref[...][:, :8].astype(f32)
    ltri = jnp.where(
        lax.broadcasted_iota(jnp.int32, (8, 8), 0)
        < lax.broadcasted_iota(jnp.int32, (8, 8), 1), 1.0, 0.0)
    base = lax.dot_general(cnt8, ltri, (((1,), (0,)), ((), ())),
                           preferred_element_type=f32, precision=hi)
    sel8 = jnp.minimum(jnp.maximum(float(_K) - base, 0.0), cnt8)
    tabs = lax.dot_general(p_ref[...][:, :_V], tab_ref[...],
                           (((1,), (0,)), ((), ())),
                           preferred_element_type=f32, precision=hi)
    query = lax.dot_general(cnt8, tabs, (((1,), (0,)), ((), ())),
                            preferred_element_type=f32, precision=hi) * (1.0 / _S)
    qa = lax.dot_general(query, wq, (((1,), (1,)), ((), ())),
                         preferred_element_type=f32, precision=hi)
    tb = lax.dot_general(tabs, wsum, (((1,), (1,)), ((), ())),
                         preferred_element_type=f32, precision=hi)
    m1 = jnp.maximum(qa[:, None, :] + tb[None, :, :] + bm1_ref[...], 0.0)
    m1 = m1.reshape(_B * 8, _BOT)
    m2 = lax.dot_general(m1, wm2_ref[...], (((1,), (1,)), ((), ())),
                         preferred_element_type=f32, precision=lax.Precision.HIGHEST)
    m2 = jnp.maximum(m2 + bm2_ref[...], 0.0)
    m3 = lax.dot_general(m2, wm3_ref[...], (((1,), (1,)), ((), ())),
                         preferred_element_type=f32, precision=lax.Precision.HIGHEST) + bm3_ref[...]
    m3 = m3.reshape(_B, 8, _D)
    approx = jnp.sum(sel8[:, :, None] * m3, axis=1) * (1.0 / _K)
    c = lax.dot_general(approx, wc1_ref[...], (((1,), (1,)), ((), ())),
                        preferred_element_type=f32, precision=lax.Precision.HIGHEST)
    c = jnp.maximum(c + bc1_ref[...], 0.0)
    logit = jnp.sum(c * wc2_ref[...], axis=1, keepdims=True) + bc2_ref[...]
    out_ref[...] = _sigmoid(logit)


@functools.lru_cache(maxsize=1)
def _make_sc_kernel():
    mesh = plsc.VectorSubcoreMesh(core_axis_name="c", subcore_axis_name="s")

    @functools.partial(
        pl.kernel,
        mesh=mesh,
        compiler_params=pltpu.CompilerParams(needs_layout_passes=False),
        out_type=[
            jax.ShapeDtypeStruct((_B, _S), jnp.float32),
            jax.ShapeDtypeStruct((_B, _KPAD), jnp.int32),
            jax.ShapeDtypeStruct((_B, _LANES), jnp.int32),
        ],
        scratch_types=[
            pltpu.VMEM((_S,), jnp.int32),
            pltpu.VMEM((_S,), jnp.float32),
            pltpu.VMEM((_S,), jnp.int32),
            pltpu.VMEM((_KPAD,), jnp.int32),
            pltpu.VMEM((_S,), jnp.int32),
            pltpu.VMEM((_LANES,), jnp.float32),
            pltpu.VMEM((_LANES,), jnp.int32),
            pltpu.VMEM((_LANES,), jnp.int32),
        ],
    )
    def sc_kernel(x_hbm, stab_hbm, vslot_hbm,
                  scores_hbm, idx_hbm, cnt_hbm,
                  x_v, sc_v, g_v, idx_v, hc_v, stab_v, vslot_v, cnt_v):
        pltpu.sync_copy(stab_hbm.at[0], stab_v)
        pltpu.sync_copy(vslot_hbm.at[0], vslot_v)
        wid = lax.axis_index("s") * 2 + lax.axis_index("c")
        iota = lax.iota(jnp.int32, _LANES)
        zero = jnp.zeros((_LANES,), jnp.int32)
        UNROLL = 8

        for bl in range(_BPW):
            b = wid * _BPW + bl
            pltpu.sync_copy(x_hbm.at[b], x_v)

            def pass1(c, carry):
                offs = [(c * UNROLL + u) * _LANES for u in range(UNROLL)]
                for off in offs:
                    hc_v[pl.ds(off, _LANES)] = zero
                xs = [x_v[pl.ds(off, _LANES)] for off in offs]
                ss = [plsc.load_gather(stab_v, [x16]) for x16 in xs]
                gs = [plsc.load_gather(vslot_v, [x16]) for x16 in xs]
                for off, s16 in zip(offs, ss):
                    sc_v[pl.ds(off, _LANES)] = s16
                for off, g16 in zip(offs, gs):
                    g_v[pl.ds(off, _LANES)] = g16
                scans = [plsc.scan_count(g16) for g16 in gs]
                for off, g16, (dup, lastm) in zip(offs, gs, scans):
                    plsc.store_scatter(hc_v, [off + g16], dup, mask=lastm)
                return carry

            lax.fori_loop(0, _CHUNKS // UNROLL, pass1, 0)

            def prefix(c, run):
                offs = [(c * UNROLL + u) * _LANES for u in range(UNROLL)]
                hs = [hc_v[pl.ds(off, _LANES)] for off in offs]
                for off, h in zip(offs, hs):
                    hc_v[pl.ds(off, _LANES)] = run
                    run = run + h
                return run

            tot = lax.fori_loop(0, _CHUNKS // UNROLL, prefix, zero)
            base = plsc.cumsum(tot) - tot

            def foldbase(c, carry):
                offs = [(c * UNROLL + u) * _LANES for u in range(UNROLL)]
                hs = [hc_v[pl.ds(off, _LANES)] for off in offs]
                for off, h in zip(offs, hs):
                    hc_v[pl.ds(off, _LANES)] = h + base
                return carry

            lax.fori_loop(0, _CHUNKS // UNROLL, foldbase, 0)

            def pass2(c, carry):
                offs = [(c * UNROLL + u) * _LANES for u in range(UNROLL)]
                gs = [g_v[pl.ds(off, _LANES)] for off in offs]
                scans = [plsc.scan_count(g16) for g16 in gs]
                priors = [plsc.load_gather(hc_v, [off + g16])
                          for off, g16 in zip(offs, gs)]
                for off, (dup, _), prior in zip(offs, scans, priors):
                    rank = prior + dup - 1
                    pos = off + iota
                    plsc.store_scatter(idx_v, [rank], pos, mask=rank < _K)
                return carry

            lax.fori_loop(0, _CHUNKS // UNROLL, pass2, 0)

            pltpu.sync_copy(sc_v, scores_hbm.at[b])
            pltpu.sync_copy(idx_v, idx_hbm.at[b])
            cnt_v[...] = tot
            pltpu.sync_copy(cnt_v, cnt_hbm.at[b])

    return sc_kernel


def _sc_call(x32, stab16, usc16):
    return _make_sc_kernel()(x32, stab16, usc16)


def kernel(x, table, W1, b1, W2, b2, Wm1, bm1, Wm2, bm2, Wm3, bm3,
           Wc1, bc1, Wc2, bc2):
    f32 = jnp.float32
    x32 = x.astype(jnp.int32)

    stab16, vslot16, pmat = pl.pallas_call(
        _scorer_body,
        out_shape=[
            jax.ShapeDtypeStruct((1, _LANES), f32),
            jax.ShapeDtypeStruct((1, _LANES), jnp.int32),
            jax.ShapeDtypeStruct((8, 8), f32),
        ],
    )(table, W1, b1, W2, b2)

    scores, idx_pad, cnt_slot = _sc_call(x32, stab16, vslot16)
    idx = idx_pad[:, :_K]

    out_mat = pl.pallas_call(
        _finish_body,
        out_shape=jax.ShapeDtypeStruct((_B, 1), f32),
    )(cnt_slot, pmat, table, Wm1, bm1, Wm2, bm2, Wm3, bm3, Wc1, bc1, Wc2, bc2)
    out = out_mat.reshape(_B)
    return (out, idx, scores)


# speedup vs baseline: 1.5706x; 1.0554x over previous
import functools

import jax
import jax.numpy as jnp
from jax import lax
from jax.experimental import pallas as pl
from jax.experimental.pallas import tpu as pltpu
from jax.experimental.pallas import tpu_sc as plsc

_B = 64
_S = 4096
_D = 256
_V = 6
_BOT = 8
_K = max(1, int(_S * 0.1))
_KPAD = 416
_LANES = 16
_CHUNKS = _S // _LANES
_NWORKERS = 32
_BPW = _B // _NWORKERS


def _sigmoid(x):
    return 1.0 / (1.0 + jnp.exp(-x))


def _scorer_body(tab_ref, w1_ref, b1_ref, w2_ref, b2_ref,
                 stab_ref, vslot_ref, p_ref):
    f32 = jnp.float32
    h = lax.dot_general(tab_ref[...], w1_ref[...], (((1,), (1,)), ((), ())),
                        preferred_element_type=f32, precision=lax.Precision.HIGHEST)
    h = jnp.maximum(h + b1_ref[...], 0.0)
    logit = jnp.sum(h * w2_ref[...], axis=1, keepdims=True) + b2_ref[...]
    scol = _sigmoid(logit)
    scol = jnp.concatenate([scol, jnp.zeros((8 - _V, 1), f32)], axis=0)
    eye = jnp.eye(8, dtype=f32)
    tr = lambda col: lax.dot_general(col, eye, (((0,), (0,)), ((), ())),
                                     preferred_element_type=f32,
                                     precision=lax.Precision.HIGHEST)
    srow = tr(scol)
    scol_b = jnp.broadcast_to(scol, (8, 8))
    srow_b = jnp.broadcast_to(srow, (8, 8))
    lane = lax.broadcasted_iota(jnp.int32, (8, 8), 1)
    subl = lax.broadcasted_iota(jnp.int32, (8, 8), 0)
    validj = lane < _V
    eqlow = (srow_b == scol_b) & (lane < subl) & validj
    first_col = jnp.sum(jnp.where(eqlow, 1.0, 0.0), axis=1, keepdims=True) == 0.0
    first_row = tr(jnp.where(first_col, 1.0, 0.0))
    first_row_b = jnp.broadcast_to(first_row, (8, 8))
    gt = (srow_b > scol_b) & validj
    gid_col = jnp.sum(jnp.where(gt, 1.0, 0.0) * first_row_b, axis=1, keepdims=True)
    gid_row = tr(gid_col)
    gid_row_b = jnp.broadcast_to(gid_row, (8, 8))
    p_ref[...] = jnp.where(
        (gid_row_b == subl.astype(f32)) & (first_row_b > 0.0) & validj, 1.0, 0.0)
    pad8 = jnp.zeros((1, 8), f32)
    stab_ref[...] = jnp.concatenate([srow, pad8], axis=1)
    vslot_ref[...] = jnp.concatenate(
        [gid_row, pad8], axis=1).astype(jnp.int32)


def _finish_body(cnt_ref, p_ref, tab_ref, wm1_ref, bm1_ref,
                 wm2_ref, bm2_ref, wm3_ref, bm3_ref, wc1_ref, bc1_ref,
                 wc2_ref, bc2_ref, out_ref):
    f32 = jnp.float32
    hi = lax.Precision.HIGHEST
    wm1 = wm1_ref[...]
    wq = wm1[:, :_D]
    wsum = wm1[:, _D:2 * _D] + wm1[:, 2 * _D:]
    cnt8 = cnt_ref[...][:, :8].astype(f32)
    ltri = jnp.where(
        lax.broadcasted_iota(jnp.int32, (8, 8), 0)
        < lax.broadcasted_iota(jnp.int32, (8, 8), 1), 1.0, 0.0)
    base = lax.dot_general(cnt8, ltri, (((1,), (0,)), ((), ())),
                           preferred_element_type=f32, precision=hi)
    sel8 = jnp.minimum(jnp.maximum(float(_K) - base, 0.0), cnt8)
    tabs = lax.dot_general(p_ref[...][:, :_V], tab_ref[...],
                           (((1,), (0,)), ((), ())),
                           preferred_element_type=f32, precision=hi)
    query = lax.dot_general(cnt8, tabs, (((1,), (0,)), ((), ())),
                            preferred_element_type=f32, precision=hi) * (1.0 / _S)
    qa = lax.dot_general(query, wq, (((1,), (1,)), ((), ())),
                         preferred_element_type=f32, precision=hi)
    tb = lax.dot_general(tabs, wsum, (((1,), (1,)), ((), ())),
                         preferred_element_type=f32, precision=hi)
    m1 = jnp.maximum(qa[:, None, :] + tb[None, :, :] + bm1_ref[...], 0.0)
    m1 = m1.reshape(_B * 8, _BOT)
    m2 = lax.dot_general(m1, wm2_ref[...], (((1,), (1,)), ((), ())),
                         preferred_element_type=f32, precision=lax.Precision.HIGHEST)
    m2 = jnp.maximum(m2 + bm2_ref[...], 0.0)
    m3 = lax.dot_general(m2, wm3_ref[...], (((1,), (1,)), ((), ())),
                         preferred_element_type=f32, precision=lax.Precision.HIGHEST) + bm3_ref[...]
    m3 = m3.reshape(_B, 8, _D)
    approx = jnp.sum(sel8[:, :, None] * m3, axis=1) * (1.0 / _K)
    c = lax.dot_general(approx, wc1_ref[...], (((1,), (1,)), ((), ())),
                        preferred_element_type=f32, precision=lax.Precision.HIGHEST)
    c = jnp.maximum(c + bc1_ref[...], 0.0)
    logit = jnp.sum(c * wc2_ref[...], axis=1, keepdims=True) + bc2_ref[...]
    out_ref[...] = _sigmoid(logit)


@functools.lru_cache(maxsize=1)
def _make_sc_kernel():
    mesh = plsc.VectorSubcoreMesh(core_axis_name="c", subcore_axis_name="s")

    @functools.partial(
        pl.kernel,
        mesh=mesh,
        compiler_params=pltpu.CompilerParams(needs_layout_passes=False),
        out_type=[
            jax.ShapeDtypeStruct((_B, _S), jnp.float32),
            jax.ShapeDtypeStruct((_B, _KPAD), jnp.int32),
            jax.ShapeDtypeStruct((_B, _LANES), jnp.int32),
        ],
        scratch_types=[
            pltpu.VMEM((_BPW * _S,), jnp.int32),
            pltpu.VMEM((_BPW * _S,), jnp.float32),
            pltpu.VMEM((_S,), jnp.int32),
            pltpu.VMEM((_KPAD,), jnp.int32),
            pltpu.VMEM((_KPAD,), jnp.int32),
            pltpu.VMEM((_S,), jnp.int32),
            pltpu.VMEM((_LANES,), jnp.float32),
            pltpu.VMEM((_LANES,), jnp.int32),
            pltpu.VMEM((_LANES,), jnp.int32),
            pltpu.VMEM((_LANES,), jnp.int32),
            pltpu.SemaphoreType.DMA,
            pltpu.SemaphoreType.DMA,
        ],
    )
    def sc_kernel(x_hbm, stab_hbm, vslot_hbm,
                  scores_hbm, idx_hbm, cnt_hbm,
                  x2_v, sc2_v, g_v, idx_va, idx_vb, hc_v, stab_v, vslot_v,
                  cnt_va, cnt_vb, sem_in, sem_out):
        wid = lax.axis_index("s") * 2 + lax.axis_index("c")
        cpx = [pltpu.async_copy(x_hbm.at[wid * _BPW + bl],
                                x2_v.at[pl.ds(bl * _S, _S)], sem_in)
               for bl in range(_BPW)]
        pltpu.sync_copy(stab_hbm.at[0], stab_v)
        pltpu.sync_copy(vslot_hbm.at[0], vslot_v)
        iota = lax.iota(jnp.int32, _LANES)
        zero = jnp.zeros((_LANES,), jnp.int32)
        UNROLL = 8
        outcps = []

        for bl in range(_BPW):
            b = wid * _BPW + bl
            xo = bl * _S
            idx_v = idx_va if bl == 0 else idx_vb
            cnt_v = cnt_va if bl == 0 else cnt_vb
            cpx[bl].wait()

            def pass1(c, carry):
                offs = [(c * UNROLL + u) * _LANES for u in range(UNROLL)]
                for off in offs:
                    hc_v[pl.ds(off, _LANES)] = zero
                xs = [x2_v[pl.ds(xo + off, _LANES)] for off in offs]
                ss = [plsc.load_gather(stab_v, [x16]) for x16 in xs]
                gs = [plsc.load_gather(vslot_v, [x16]) for x16 in xs]
                for off, s16 in zip(offs, ss):
                    sc2_v[pl.ds(xo + off, _LANES)] = s16
                for off, g16 in zip(offs, gs):
                    g_v[pl.ds(off, _LANES)] = g16
                scans = [plsc.scan_count(g16) for g16 in gs]
                for off, g16, (dup, lastm) in zip(offs, gs, scans):
                    plsc.store_scatter(hc_v, [off + g16], dup, mask=lastm)
                return carry

            lax.fori_loop(0, _CHUNKS // UNROLL, pass1, 0)

            def prefix(c, run):
                offs = [(c * UNROLL + u) * _LANES for u in range(UNROLL)]
                hs = [hc_v[pl.ds(off, _LANES)] for off in offs]
                for off, h in zip(offs, hs):
                    hc_v[pl.ds(off, _LANES)] = run
                    run = run + h
                return run

            tot = lax.fori_loop(0, _CHUNKS // UNROLL, prefix, zero)
            base = plsc.cumsum(tot) - tot

            def foldbase(c, carry):
                offs = [(c * UNROLL + u) * _LANES for u in range(UNROLL)]
                hs = [hc_v[pl.ds(off, _LANES)] for off in offs]
                for off, h in zip(offs, hs):
                    hc_v[pl.ds(off, _LANES)] = h + base
                return carry

            lax.fori_loop(0, _CHUNKS // UNROLL, foldbase, 0)

            def pass2(c, carry):
                offs = [(c * UNROLL + u) * _LANES for u in range(UNROLL)]
                gs = [g_v[pl.ds(off, _LANES)] for off in offs]
                scans = [plsc.scan_count(g16) for g16 in gs]
                priors = [plsc.load_gather(hc_v, [off + g16])
                          for off, g16 in zip(offs, gs)]
                for off, (dup, _), prior in zip(offs, scans, priors):
                    rank = prior + dup - 1
                    pos = off + iota
                    plsc.store_scatter(idx_v, [rank], pos, mask=rank < _K)
                return carry

            lax.fori_loop(0, _CHUNKS // UNROLL, pass2, 0)

            cnt_v[...] = tot
            outcps.append(pltpu.async_copy(
                sc2_v.at[pl.ds(xo, _S)], scores_hbm.at[b], sem_out))
            outcps.append(pltpu.async_copy(idx_v, idx_hbm.at[b], sem_out))
            outcps.append(pltpu.async_copy(cnt_v, cnt_hbm.at[b], sem_out))

        for cp in outcps:
            cp.wait()

    return sc_kernel


def _sc_call(x32, stab16, usc16):
    return _make_sc_kernel()(x32, stab16, usc16)


def kernel(x, table, W1, b1, W2, b2, Wm1, bm1, Wm2, bm2, Wm3, bm3,
           Wc1, bc1, Wc2, bc2):
    f32 = jnp.float32
    x32 = x.astype(jnp.int32)

    stab16, vslot16, pmat = pl.pallas_call(
        _scorer_body,
        out_shape=[
            jax.ShapeDtypeStruct((1, _LANES), f32),
            jax.ShapeDtypeStruct((1, _LANES), jnp.int32),
            jax.ShapeDtypeStruct((8, 8), f32),
        ],
    )(table, W1, b1, W2, b2)

    scores, idx_pad, cnt_slot = _sc_call(x32, stab16, vslot16)
    idx = idx_pad[:, :_K]

    out_mat = pl.pallas_call(
        _finish_body,
        out_shape=jax.ShapeDtypeStruct((_B, 1), f32),
    )(cnt_slot, pmat, table, Wm1, bm1, Wm2, bm2, Wm3, bm3, Wc1, bc1, Wc2, bc2)
    out = out_mat.reshape(_B)
    return (out, idx, scores)
